# speedup vs baseline: 16.6912x; 16.6912x over previous
import functools

import jax
import jax.numpy as jnp
from jax import lax
from jax.experimental import pallas as pl

_B, _N, _M = 16, 4096, 1024
_C1, _C2 = 256, 512
_O1, _O2 = 512, 256
_NTOT = _B * _N
_NBLK_A = 256
_NBLK = 512


def _gt_body(kf_ref, w1a_ref, gt_ref):
    gt_ref[...] = lax.dot_general(
        kf_ref[...], w1a_ref[...], (((0,), (1,)), ((), ())),
        preferred_element_type=jnp.float32)


def _knn_body(ut_ref, kt_ref, gt_ref, out_ref):
    u = ut_ref[...]
    k = kt_ref[...]
    uu = jnp.sum(u * u, axis=0)[:, None]
    kk = jnp.sum(k * k, axis=0)[None, :]
    cross = lax.dot_general(u, k, (((0,), (0,)), ((), ())),
                            preferred_element_type=jnp.float32)
    d = jnp.maximum(uu + kk - 2.0 * cross, 0.0)
    iota = lax.broadcasted_iota(jnp.int32, (_NBLK_A, _M), 1)
    recips, imins = [], []
    dcur = d
    for _ in range(3):
        vmin = jnp.min(dcur, axis=1, keepdims=True)
        imin = jnp.min(jnp.where(dcur == vmin, iota, _M), axis=1, keepdims=True)
        recips.append(1.0 / (jnp.sqrt(vmin) + 1e-8))
        imins.append(imin)
        dcur = jnp.where(iota == imin, jnp.float32(jnp.inf), dcur)
    norm = recips[0] + recips[1] + recips[2]
    wacc = jnp.zeros((_NBLK_A, _M), jnp.float32)
    for rk, ik in zip(recips, imins):
        wacc = wacc + jnp.where(iota == ik, rk / norm, 0.0)
    out_ref[...] = lax.dot_general(wacc, gt_ref[...], (((1,), (0,)), ((), ())),
                                   preferred_element_type=jnp.float32)


def _e_body(interp_ref, uf_ref, w1b_ref, b1_ref, y1_ref, st_ref):
    y = lax.dot_general(uf_ref[...], w1b_ref[...], (((0,), (1,)), ((), ())),
                        preferred_element_type=jnp.float32)
    y = y + interp_ref[...] + b1_ref[...]
    y1_ref[...] = y
    s = jnp.sum(y, axis=0, keepdims=True)
    s2 = jnp.sum(y * y, axis=0, keepdims=True)

    @pl.when(jnp.logical_and(pl.program_id(0) == 0, pl.program_id(1) == 0))
    def _():
        st_ref[...] = jnp.zeros_like(st_ref)

    st_ref[...] = st_ref[...] + jnp.concatenate(
        [s, s2, jnp.zeros((6, _O1), jnp.float32)], axis=0)


def _f_body(y1_ref, st1_ref, g1_ref, bt1_ref, w2_ref, b2_ref, y2_ref, st_ref):
    st = st1_ref[...]
    mean = st[0:1, :] / _NTOT
    var = st[1:2, :] / _NTOT - mean * mean
    inv = lax.rsqrt(var + 1e-5) * g1_ref[...]
    h = jnp.maximum((y1_ref[...] - mean) * inv + bt1_ref[...], 0.0)
    y2 = lax.dot_general(w2_ref[...], h, (((1,), (1,)), ((), ())),
                         preferred_element_type=jnp.float32)
    y2 = y2 + b2_ref[...]
    y2_ref[...] = y2
    s = jnp.sum(y2, axis=1, keepdims=True)
    s2 = jnp.sum(y2 * y2, axis=1, keepdims=True)

    @pl.when(pl.program_id(0) == 0)
    def _():
        st_ref[...] = jnp.zeros_like(st_ref)

    st_ref[...] = st_ref[...] + jnp.concatenate(
        [s, s2, jnp.zeros((_O2, 6), jnp.float32)], axis=1)


def _g_body(y2_ref, st2_ref, g2_ref, bt2_ref, out_ref):
    st = st2_ref[...]
    mean = st[:, 0:1] / _NTOT
    var = st[:, 1:2] / _NTOT - mean * mean
    inv = lax.rsqrt(var + 1e-5) * g2_ref[...]
    out_ref[...] = jnp.maximum((y2_ref[...] - mean) * inv + bt2_ref[...], 0.0)


def kernel(unknown, known, unknow_feats, known_feats,
           W1, b1, g1, bt1, W2, b2, g2, bt2):
    f32 = jnp.float32
    ut8 = jnp.concatenate(
        [jnp.transpose(unknown, (0, 2, 1)), jnp.zeros((_B, 5, _N), f32)], axis=1)
    kt8 = jnp.concatenate(
        [jnp.transpose(known, (0, 2, 1)), jnp.zeros((_B, 5, _M), f32)], axis=1)
    W1a = W1[:, :_C2]
    W1b = W1[:, _C2:]
    b1r = b1.reshape(1, _O1)
    g1r = g1.reshape(1, _O1)
    bt1r = bt1.reshape(1, _O1)
    b2r = b2.reshape(_O2, 1)
    g2r = g2.reshape(_O2, 1)
    bt2r = bt2.reshape(_O2, 1)

    gt = pl.pallas_call(
        _gt_body,
        grid=(_B,),
        in_specs=[
            pl.BlockSpec((None, _C2, _M), lambda b: (b, 0, 0)),
            pl.BlockSpec((_O1, _C2), lambda b: (0, 0)),
        ],
        out_specs=pl.BlockSpec((None, _M, _O1), lambda b: (b, 0, 0)),
        out_shape=jax.ShapeDtypeStruct((_B, _M, _O1), f32),
    )(known_feats, W1a)

    nja = _N // _NBLK_A
    interp = pl.pallas_call(
        _knn_body,
        grid=(_B, nja),
        in_specs=[
            pl.BlockSpec((None, 8, _NBLK_A), lambda b, j: (b, 0, j)),
            pl.BlockSpec((None, 8, _M), lambda b, j: (b, 0, 0)),
            pl.BlockSpec((None, _M, _O1), lambda b, j: (b, 0, 0)),
        ],
        out_specs=pl.BlockSpec((None, _NBLK_A, _O1), lambda b, j: (b, j, 0)),
        out_shape=jax.ShapeDtypeStruct((_B, _N, _O1), f32),
    )(ut8, kt8, gt)

    nj = _N // _NBLK
    y1, st1 = pl.pallas_call(
        _e_body,
        grid=(_B, nj),
        in_specs=[
            pl.BlockSpec((None, _NBLK, _O1), lambda b, j: (b, j, 0)),
            pl.BlockSpec((None, _C1, _NBLK), lambda b, j: (b, 0, j)),
            pl.BlockSpec((_O1, _C1), lambda b, j: (0, 0)),
            pl.BlockSpec((1, _O1), lambda b, j: (0, 0)),
        ],
        out_specs=[
            pl.BlockSpec((None, _NBLK, _O1), lambda b, j: (b, j, 0)),
            pl.BlockSpec((8, _O1), lambda b, j: (0, 0)),
        ],
        out_shape=[
            jax.ShapeDtypeStruct((_B, _N, _O1), f32),
            jax.ShapeDtypeStruct((8, _O1), f32),
        ],
    )(interp, unknow_feats, W1b, b1r)

    nt = _NTOT // _NBLK
    y2, st2 = pl.pallas_call(
        _f_body,
        grid=(nt,),
        in_specs=[
            pl.BlockSpec((None, _NBLK, _O1), lambda t: (t // nj, t % nj, 0)),
            pl.BlockSpec((8, _O1), lambda t: (0, 0)),
            pl.BlockSpec((1, _O1), lambda t: (0, 0)),
            pl.BlockSpec((1, _O1), lambda t: (0, 0)),
            pl.BlockSpec((_O2, _O1), lambda t: (0, 0)),
            pl.BlockSpec((_O2, 1), lambda t: (0, 0)),
        ],
        out_specs=[
            pl.BlockSpec((None, _O2, _NBLK), lambda t: (t // nj, 0, t % nj)),
            pl.BlockSpec((_O2, 8), lambda t: (0, 0)),
        ],
        out_shape=[
            jax.ShapeDtypeStruct((_B, _O2, _N), f32),
            jax.ShapeDtypeStruct((_O2, 8), f32),
        ],
    )(y1, st1, g1r, bt1r, W2, b2r)

    out = pl.pallas_call(
        _g_body,
        grid=(_B, nj),
        in_specs=[
            pl.BlockSpec((None, _O2, _NBLK), lambda b, j: (b, 0, j)),
            pl.BlockSpec((_O2, 8), lambda b, j: (0, 0)),
            pl.BlockSpec((_O2, 1), lambda b, j: (0, 0)),
            pl.BlockSpec((_O2, 1), lambda b, j: (0, 0)),
        ],
        out_specs=pl.BlockSpec((None, _O2, _NBLK), lambda b, j: (b, 0, j)),
        out_shape=jax.ShapeDtypeStruct((_B, _O2, _N), f32),
    )(y2, st2, g2r, bt2r)
    return out
